# baseline (speedup 1.0000x reference)
import functools

import jax
import jax.numpy as jnp
from jax.experimental import pallas as pl
from jax.experimental.pallas import tpu as pltpu

_EPS = 1e-8
_NEG = -1e30


def _attn_body(fm_ref, lt_ref, bias_ref, qflag_ref, o_ref, kn_ref, ltb_ref, *, bq, c):
    i = pl.program_id(1)

    @pl.when(i == 0)
    def _init_kn():
        lt = lt_ref[0]
        kn = lt / (jnp.sqrt(jnp.sum(lt * lt, axis=0, keepdims=True)) + _EPS)
        kn_ref[...] = kn.astype(jnp.bfloat16)
        ltb_ref[...] = lt.astype(jnp.bfloat16)

    fm = fm_ref[0]
    qn = fm / (jnp.sqrt(jnp.sum(fm * fm, axis=0, keepdims=True)) + _EPS)

    scores = jax.lax.dot_general(
        qn.astype(jnp.bfloat16), kn_ref[...], (((0,), (0,)), ((), ())),
        preferred_element_type=jnp.float32)
    p = jnp.exp(scores + bias_ref[0][None, :])
    s = jnp.sum(p, axis=1)

    out_t = jax.lax.dot_general(
        ltb_ref[...], p.astype(jnp.bfloat16), (((1,), (1,)), ((), ())),
        preferred_element_type=jnp.float32)
    scale = qflag_ref[0] / s
    o_ref[0, 2 * c:, :] = out_t * scale[None, :]
    o_ref[0, :c, :] = fm
    o_ref[0, c:2 * c, :] = lt_ref[0, :, pl.ds(i * bq, bq)]


def _shift_concat(inp_chw, bias, flag_f, *, bq):
    B, C, HW = inp_chw.shape
    c = C // 2
    grid = (B, HW // bq)
    return pl.pallas_call(
        functools.partial(_attn_body, bq=bq, c=c),
        grid=grid,
        in_specs=[
            pl.BlockSpec((1, c, bq), lambda b, i: (b, 0, i)),
            pl.BlockSpec((1, c, HW), lambda b, i: (b, 1, 0)),
            pl.BlockSpec((1, HW), lambda b, i: (0, 0)),
            pl.BlockSpec((1, bq), lambda b, i: (0, i)),
        ],
        out_specs=pl.BlockSpec((1, 3 * c, bq), lambda b, i: (b, 0, i)),
        out_shape=jax.ShapeDtypeStruct((B, 3 * c, HW), jnp.float32),
        scratch_shapes=[pltpu.VMEM((c, HW), jnp.bfloat16),
                        pltpu.VMEM((c, HW), jnp.bfloat16)],
        compiler_params=pltpu.CompilerParams(
            dimension_semantics=("arbitrary", "arbitrary"),
        ),
    )(inp_chw, inp_chw, bias, flag_f)


def kernel(input, mask, shift_sz, stride, triple_w, flag):
    B, C, H, W = input.shape
    HW = H * W
    flag_f = flag.astype(jnp.float32).reshape(1, HW)
    bias = flag_f * _NEG
    out = _shift_concat(input.reshape(B, C, HW), bias, flag_f, bq=min(256, HW))
    return out.reshape(B, C + C // 2, H, W)


# speedup vs baseline: 1.0875x; 1.0875x over previous
import functools

import jax
import jax.numpy as jnp
from jax.experimental import pallas as pl
from jax.experimental.pallas import tpu as pltpu

_EPS = 1e-8
_NEG = -1e30


def _attn_body(fm_ref, lt_ref, bias_ref, qflag_ref, o_ref, kn_ref, ltb_ref, *, bq, c):
    i = pl.program_id(1)

    @pl.when(i == 0)
    def _init_kn():
        lt = lt_ref[0]
        kn = lt / (jnp.sqrt(jnp.sum(lt * lt, axis=0, keepdims=True)) + _EPS)
        kn_ref[...] = kn.astype(jnp.bfloat16)
        ltb_ref[...] = lt.astype(jnp.bfloat16)

    fm = fm_ref[0]
    qn = fm / (jnp.sqrt(jnp.sum(fm * fm, axis=0, keepdims=True)) + _EPS)

    scores = jax.lax.dot_general(
        qn.astype(jnp.bfloat16), kn_ref[...], (((0,), (0,)), ((), ())),
        preferred_element_type=jnp.float32)
    p = jnp.exp(scores + bias_ref[0][None, :])
    s = jnp.sum(p, axis=1)

    out_t = jax.lax.dot_general(
        ltb_ref[...], p.astype(jnp.bfloat16), (((1,), (1,)), ((), ())),
        preferred_element_type=jnp.float32)
    scale = qflag_ref[0] / s
    o_ref[0, 2 * c:, :] = out_t * scale[None, :]
    o_ref[0, :c, :] = fm
    o_ref[0, c:2 * c, :] = lt_ref[0, :, pl.ds(i * bq, bq)]


def _shift_concat(inp_chw, bias, flag_f, *, bq):
    B, C, HW = inp_chw.shape
    c = C // 2
    grid = (B, HW // bq)
    return pl.pallas_call(
        functools.partial(_attn_body, bq=bq, c=c),
        grid=grid,
        in_specs=[
            pl.BlockSpec((1, c, bq), lambda b, i: (b, 0, i)),
            pl.BlockSpec((1, c, HW), lambda b, i: (b, 1, 0)),
            pl.BlockSpec((1, HW), lambda b, i: (0, 0)),
            pl.BlockSpec((1, bq), lambda b, i: (0, i)),
        ],
        out_specs=pl.BlockSpec((1, 3 * c, bq), lambda b, i: (b, 0, i)),
        out_shape=jax.ShapeDtypeStruct((B, 3 * c, HW), jnp.float32),
        scratch_shapes=[pltpu.VMEM((c, HW), jnp.bfloat16),
                        pltpu.VMEM((c, HW), jnp.bfloat16)],
        compiler_params=pltpu.CompilerParams(
            dimension_semantics=("arbitrary", "arbitrary"),
        ),
    )(inp_chw, inp_chw, bias, flag_f)


def kernel(input, mask, shift_sz, stride, triple_w, flag):
    B, C, H, W = input.shape
    HW = H * W
    flag_f = flag.astype(jnp.float32).reshape(1, HW)
    bias = flag_f * _NEG
    out = _shift_concat(input.reshape(B, C, HW), bias, flag_f, bq=min(1024, HW))
    return out.reshape(B, C + C // 2, H, W)


# speedup vs baseline: 1.0910x; 1.0033x over previous
import functools

import jax
import jax.numpy as jnp
from jax.experimental import pallas as pl
from jax.experimental.pallas import tpu as pltpu

_EPS = 1e-8
_NEG = -1e30


def _attn_body(fm_ref, lt_ref, bias_ref, qflag_ref, o_ref, kn_ref, ltb_ref, *, bq, c):
    i = pl.program_id(1)

    @pl.when(i == 0)
    def _init_kn():
        lt = lt_ref[0]
        kn = lt / (jnp.sqrt(jnp.sum(lt * lt, axis=0, keepdims=True)) + _EPS)
        kn_ref[...] = kn.astype(jnp.bfloat16)
        ltb_ref[...] = lt.astype(jnp.bfloat16)

    fm = fm_ref[0]
    qn = fm / (jnp.sqrt(jnp.sum(fm * fm, axis=0, keepdims=True)) + _EPS)

    scores = jax.lax.dot_general(
        qn.astype(jnp.bfloat16), kn_ref[...], (((0,), (0,)), ((), ())),
        preferred_element_type=jnp.float32)
    p = jnp.exp(scores + bias_ref[0][None, :])
    s = jnp.sum(p, axis=1)

    out_t = jax.lax.dot_general(
        ltb_ref[...], p.astype(jnp.bfloat16), (((1,), (1,)), ((), ())),
        preferred_element_type=jnp.float32)
    scale = qflag_ref[0] / s
    o_ref[0, 2 * c:, :] = out_t * scale[None, :]
    o_ref[0, :c, :] = fm
    o_ref[0, c:2 * c, :] = lt_ref[0, :, pl.ds(i * bq, bq)]


def _shift_concat(inp_chw, bias, flag_f, *, bq):
    B, C, HW = inp_chw.shape
    c = C // 2
    grid = (B, HW // bq)
    return pl.pallas_call(
        functools.partial(_attn_body, bq=bq, c=c),
        grid=grid,
        in_specs=[
            pl.BlockSpec((1, c, bq), lambda b, i: (b, 0, i)),
            pl.BlockSpec((1, c, HW), lambda b, i: (b, 1, 0)),
            pl.BlockSpec((1, HW), lambda b, i: (0, 0)),
            pl.BlockSpec((1, bq), lambda b, i: (0, i)),
        ],
        out_specs=pl.BlockSpec((1, 3 * c, bq), lambda b, i: (b, 0, i)),
        out_shape=jax.ShapeDtypeStruct((B, 3 * c, HW), jnp.float32),
        scratch_shapes=[pltpu.VMEM((c, HW), jnp.bfloat16),
                        pltpu.VMEM((c, HW), jnp.bfloat16)],
        compiler_params=pltpu.CompilerParams(
            dimension_semantics=("arbitrary", "arbitrary"),
        ),
    )(inp_chw, inp_chw, bias, flag_f)


def kernel(input, mask, shift_sz, stride, triple_w, flag):
    B, C, H, W = input.shape
    HW = H * W
    flag_f = flag.astype(jnp.float32).reshape(1, HW)
    bias = flag_f * _NEG
    out = _shift_concat(input.reshape(B, C, HW), bias, flag_f, bq=min(2048, HW))
    return out.reshape(B, C + C // 2, H, W)
